# baseline (speedup 1.0000x reference)
import functools

import jax
import jax.numpy as jnp
from jax import lax
from jax.experimental import pallas as pl
from jax.experimental.pallas import tpu as pltpu
from jax.experimental.pallas import tpu_sc as plsc

NUM_CONCEPTS = 1000000
D = 64
B = 16384

NC, NS = 2, 16
NW = NC * NS
BPW = B // NW
CHUNK = 128
NCHUNK = BPW // CHUNK

_sc_mesh = plsc.VectorSubcoreMesh(core_axis_name="c", subcore_axis_name="s")


@functools.partial(
    pl.kernel,
    out_type=(
        jax.ShapeDtypeStruct((B, D), jnp.float32),
        jax.ShapeDtypeStruct((B, D), jnp.float32),
    ),
    mesh=_sc_mesh,
    compiler_params=pltpu.CompilerParams(use_tc_tiling_on_sc=False),
    scratch_types=[
        pltpu.VMEM((NCHUNK, CHUNK), jnp.int32),
        pltpu.VMEM((NCHUNK, CHUNK), jnp.int32),
        pltpu.VMEM((BPW, D), jnp.float32),
        pltpu.VMEM((BPW, D), jnp.float32),
        pltpu.SemaphoreType.DMA,
    ],
)
def _sc_gather(table_hbm, idx1_hbm, idx2_hbm, out1_hbm, out2_hbm,
               idx1_v, idx2_v, rows1_v, rows2_v, sem):
    wid = lax.axis_index("s") * NC + lax.axis_index("c")
    base = wid * BPW
    pltpu.sync_copy(idx1_hbm.at[pl.ds(wid * NCHUNK, NCHUNK)], idx1_v)
    pltpu.sync_copy(idx2_hbm.at[pl.ds(wid * NCHUNK, NCHUNK)], idx2_v)
    copies = []
    for j in range(NCHUNK):
        copies.append(pltpu.async_copy(
            table_hbm.at[idx1_v.at[j]],
            rows1_v.at[pl.ds(j * CHUNK, CHUNK)], sem))
        copies.append(pltpu.async_copy(
            table_hbm.at[idx2_v.at[j]],
            rows2_v.at[pl.ds(j * CHUNK, CHUNK)], sem))
    for c in copies:
        c.wait()
    pltpu.sync_copy(rows1_v, out1_hbm.at[pl.ds(base, BPW)])
    pltpu.sync_copy(rows2_v, out2_hbm.at[pl.ds(base, BPW)])


BLK = 1024


def _mlp_body(e1_ref, e2_ref, w1a_ref, w1b_ref, b1_ref, w2_ref, b2_ref,
              w3_ref, b3_ref, out_ref):
    h = jnp.dot(e1_ref[...], w1a_ref[...], preferred_element_type=jnp.float32)
    h = h + jnp.dot(e2_ref[...], w1b_ref[...],
                    preferred_element_type=jnp.float32)
    h = jnp.maximum(h + b1_ref[...], 0.0)
    h2 = jnp.dot(h, w2_ref[...], preferred_element_type=jnp.float32)
    h2 = jnp.maximum(h2 + b2_ref[...], 0.0)
    logit = jnp.sum(h2 * w3_ref[...], axis=1, keepdims=True) + b3_ref[...]
    out_ref[...] = jax.nn.sigmoid(logit)


def _mlp(e1, e2, w1a, w1b, b1, w2, b2, w3_row, b3):
    grid = (B // BLK,)
    return pl.pallas_call(
        _mlp_body,
        grid=grid,
        in_specs=[
            pl.BlockSpec((BLK, D), lambda i: (i, 0)),
            pl.BlockSpec((BLK, D), lambda i: (i, 0)),
            pl.BlockSpec((D, 256), lambda i: (0, 0)),
            pl.BlockSpec((D, 256), lambda i: (0, 0)),
            pl.BlockSpec((1, 256), lambda i: (0, 0)),
            pl.BlockSpec((256, 128), lambda i: (0, 0)),
            pl.BlockSpec((1, 128), lambda i: (0, 0)),
            pl.BlockSpec((1, 128), lambda i: (0, 0)),
            pl.BlockSpec((1, 1), lambda i: (0, 0)),
        ],
        out_specs=pl.BlockSpec((BLK, 1), lambda i: (i, 0)),
        out_shape=jax.ShapeDtypeStruct((B, 1), jnp.float32),
    )(e1, e2, w1a, w1b, b1, w2, b2, w3_row, b3)


def kernel(concept_table, W1, b1, W2, b2, W3, b3, concept1_idx, concept2_idx):
    idx1 = concept1_idx.astype(jnp.int32).reshape(NW * NCHUNK, CHUNK)
    idx2 = concept2_idx.astype(jnp.int32).reshape(NW * NCHUNK, CHUNK)
    e1, e2 = _sc_gather(concept_table, idx1, idx2)
    w1a = W1[:D]
    w1b = W1[D:]
    return _mlp(e1, e2, w1a, w1b,
                b1.reshape(1, 256), W2, b2.reshape(1, 128),
                W3.reshape(1, 128), b3.reshape(1, 1))
